# baseline (speedup 1.0000x reference)
import jax
import jax.numpy as jnp
from jax.experimental import pallas as pl
from jax.experimental.pallas import tpu as pltpu

_SIGMA = (1.0, 1.0, 1.0)
_T = 8
_CB = 4


def _gauss_matrix(n, scale):
    i = jax.lax.broadcasted_iota(jnp.int32, (n, n), 0)
    j = jax.lax.broadcasted_iota(jnp.int32, (n, n), 1)
    d = (i - j).astype(jnp.float32) * scale
    k = jnp.exp(-0.5 * d * d)
    return k / jnp.sum(k, axis=1, keepdims=True)


def _kron_gauss_eye(d, scale):
    n = d * _T
    a = jax.lax.broadcasted_iota(jnp.int32, (n, n), 0)
    b = jax.lax.broadcasted_iota(jnp.int32, (n, n), 1)
    dz = ((a // _T) - (b // _T)).astype(jnp.float32) * scale
    k = jnp.exp(-0.5 * dz * dz)
    k = jnp.where((a % _T) == (b % _T), k, 0.0)
    return k / jnp.sum(k, axis=1, keepdims=True)


def _fused_kernel(v_ref, x_ref, o_ref, p_ref):
    cb, d, h, w = x_ref.shape
    ay = _gauss_matrix(h, v_ref[2] / _SIGMA[2])
    ax = _gauss_matrix(w, v_ref[1] / _SIGMA[1])
    azk = _kron_gauss_eye(d, v_ref[0] / _SIGMA[0])

    t = jax.lax.dot_general(
        x_ref[...].reshape(cb * d * h, w), ax, (((1,), (1,)), ((), ())),
        preferred_element_type=jnp.float32).reshape(cb * d, h, w)
    for di in range(cb * d):
        p_ref[di] = jnp.dot(ay, t[di], preferred_element_type=jnp.float32)

    for cj in range(cb):
        for hb in range(h // _T):
            blk = p_ref[cj * d:(cj + 1) * d, hb * _T:(hb + 1) * _T, :]
            ob = jnp.dot(azk, blk.reshape(d * _T, w),
                         preferred_element_type=jnp.float32)
            o_ref[cj, :, hb * _T:(hb + 1) * _T, :] = ob.reshape(d, _T, w)


@jax.jit
def kernel(input_, image, v_gamma):
    c, d, h, w = input_.shape
    return pl.pallas_call(
        _fused_kernel,
        grid=(c // _CB,),
        in_specs=[
            pl.BlockSpec(memory_space=pltpu.SMEM),
            pl.BlockSpec((_CB, d, h, w), lambda ci: (ci, 0, 0, 0)),
        ],
        out_specs=pl.BlockSpec((_CB, d, h, w), lambda ci: (ci, 0, 0, 0)),
        out_shape=jax.ShapeDtypeStruct((c, d, h, w), jnp.float32),
        scratch_shapes=[pltpu.VMEM((_CB * d, h, w), jnp.float32)],
        compiler_params=pltpu.CompilerParams(
            dimension_semantics=("arbitrary",)),
    )(v_gamma, input_)


# speedup vs baseline: 1.1933x; 1.1933x over previous
import jax
import jax.numpy as jnp
from jax.experimental import pallas as pl
from jax.experimental.pallas import tpu as pltpu

_SIGMA = (1.0, 1.0, 1.0)
_T = 8
_CB = 2


def _gauss_matrix(n, scale):
    i = jax.lax.broadcasted_iota(jnp.int32, (n, n), 0)
    j = jax.lax.broadcasted_iota(jnp.int32, (n, n), 1)
    d = (i - j).astype(jnp.float32) * scale
    k = jnp.exp(-0.5 * d * d)
    return k / jnp.sum(k, axis=1, keepdims=True)


def _kron_gauss_eye(d, scale):
    n = d * _T
    a = jax.lax.broadcasted_iota(jnp.int32, (n, n), 0)
    b = jax.lax.broadcasted_iota(jnp.int32, (n, n), 1)
    dz = ((a // _T) - (b // _T)).astype(jnp.float32) * scale
    k = jnp.exp(-0.5 * dz * dz)
    k = jnp.where((a % _T) == (b % _T), k, 0.0)
    return k / jnp.sum(k, axis=1, keepdims=True)


def _fused_kernel(v_ref, x_ref, o_ref, p_ref):
    cb, d, h, w = x_ref.shape
    ay = _gauss_matrix(h, v_ref[2] / _SIGMA[2])
    ax = _gauss_matrix(w, v_ref[1] / _SIGMA[1])
    azk = _kron_gauss_eye(d, v_ref[0] / _SIGMA[0])

    t = jax.lax.dot_general(
        x_ref[...].reshape(cb * d * h, w), ax, (((1,), (1,)), ((), ())),
        preferred_element_type=jnp.float32).reshape(cb * d, h, w)
    for di in range(cb * d):
        p_ref[di] = jnp.dot(ay, t[di], preferred_element_type=jnp.float32)

    for cj in range(cb):
        for hb in range(h // _T):
            blk = p_ref[cj * d:(cj + 1) * d, hb * _T:(hb + 1) * _T, :]
            ob = jnp.dot(azk, blk.reshape(d * _T, w),
                         preferred_element_type=jnp.float32)
            o_ref[cj, :, hb * _T:(hb + 1) * _T, :] = ob.reshape(d, _T, w)


@jax.jit
def kernel(input_, image, v_gamma):
    c, d, h, w = input_.shape
    return pl.pallas_call(
        _fused_kernel,
        grid=(c // _CB,),
        in_specs=[
            pl.BlockSpec(memory_space=pltpu.SMEM),
            pl.BlockSpec((_CB, d, h, w), lambda ci: (ci, 0, 0, 0)),
        ],
        out_specs=pl.BlockSpec((_CB, d, h, w), lambda ci: (ci, 0, 0, 0)),
        out_shape=jax.ShapeDtypeStruct((c, d, h, w), jnp.float32),
        scratch_shapes=[pltpu.VMEM((_CB * d, h, w), jnp.float32)],
        compiler_params=pltpu.CompilerParams(
            dimension_semantics=("arbitrary",)),
    )(v_gamma, input_)
